# baseline (speedup 1.0000x reference)
import functools

import jax
import jax.numpy as jnp
from jax import lax
from jax.experimental import pallas as pl
from jax.experimental.pallas import tpu as pltpu
from jax.experimental.pallas import tpu_sc as plsc

VOC = 512
EMB = 128
HID = 128
HH = HID // 2
B = 16
LS = 514
LT = 513
NV = VOC + 1

TROWS = 520
NW = 32
PER_W = 264
SUB = 88
NSUB = PER_W // SUB
NTOK = NW * PER_W

CH_R = 64
GR_R = 9
CH_P = 32
GR_P = 17

def _prep_body(emb_ref, t_ref):
    row = lax.broadcasted_iota(jnp.int32, (TROWS, 1), 0)
    t_ref[...] = jnp.where(row != 0, emb_ref[...], 0.0)


def _masked_table(emb_pad):
    return pl.pallas_call(
        _prep_body,
        out_shape=jax.ShapeDtypeStruct((TROWS, EMB), jnp.float32),
    )(emb_pad)


def _sc_gather(table, idx_f, idx_b, idx_m):
    mesh = plsc.VectorSubcoreMesh(core_axis_name="c", subcore_axis_name="s")

    @functools.partial(
        pl.kernel,
        mesh=mesh,
        out_type=[
            jax.ShapeDtypeStruct((NTOK, EMB), jnp.float32),
            jax.ShapeDtypeStruct((NTOK, EMB), jnp.float32),
            jax.ShapeDtypeStruct((NTOK, EMB), jnp.float32),
        ],
        scratch_types=[
            [pltpu.VMEM((SUB,), jnp.int32) for _ in range(3)],
            [pltpu.VMEM((SUB, EMB), jnp.float32) for _ in range(3)],
            pltpu.SemaphoreType.DMA,
            pltpu.SemaphoreType.DMA,
        ],
    )
    def k(t_hbm, if_hbm, ib_hbm, im_hbm,
          xf_hbm, xb_hbm, xm_hbm, idx_v, row_v, gsem, osem):
        wid = lax.axis_index("s") * 2 + lax.axis_index("c")
        base = wid * PER_W
        tasks = []
        for s in range(NSUB):
            off = base + s * SUB
            tasks.append((if_hbm, xf_hbm, off))
            tasks.append((ib_hbm, xb_hbm, off))
            tasks.append((im_hbm, xm_hbm, off))

        def start(t, slot):
            src_idx, _, off = tasks[t]
            pltpu.sync_copy(src_idx.at[pl.ds(off, SUB)], idx_v[slot])
            return pltpu.async_copy(t_hbm.at[idx_v[slot]], row_v[slot], gsem)

        outs = []
        g = start(0, 0)
        for t in range(len(tasks)):
            slot = t % 3
            g.wait()
            if t + 1 < len(tasks):
                g = start(t + 1, (t + 1) % 3)
            _, dst, off = tasks[t]
            o = pltpu.async_copy(row_v[slot], dst.at[pl.ds(off, SUB)], osem)
            outs.append(o)
            if len(outs) >= 2:
                outs.pop(0).wait()
        for o in outs:
            o.wait()

    return k(table, idx_f, idx_b, idx_m)


def _rec_body(ef_ref, eb_ref, em_ref, wi_ref, bfb_ref, bm_ref,
              wfb_ref, wm_ref, h_out, h_s, c_s, xfb_s, xm_s):
    j = pl.program_id(0)

    @pl.when(j == 0)
    def _():
        h_s[...] = jnp.zeros_like(h_s)
        c_s[...] = jnp.zeros_like(c_s)

    for sidx, e_ref in ((0, ef_ref), (1, eb_ref)):
        emb = e_ref[...].reshape(CH_R * B, EMB)
        xs = jnp.dot(emb, wi_ref[:, 4 * HH * sidx:4 * HH * (sidx + 1)],
                     preferred_element_type=jnp.float32)
        for gate in range(4):
            c0 = HID * gate + HH * sidx
            xfb_s[:, :, c0:c0 + HH] = (
                xs[:, gate * HH:(gate + 1) * HH].reshape(CH_R, B, HH)
                + bfb_ref[:, :, c0:c0 + HH])
    emb = em_ref[...].reshape(CH_R * B, EMB)
    xm_s[...] = (jnp.dot(emb, wi_ref[:, 8 * HH:],
                         preferred_element_type=jnp.float32)
                 + bm_ref[0]).reshape(CH_R, B, 4 * HID)

    wfb = wfb_ref[...]
    wm = wm_ref[...]

    def step(k, carry):
        hfb, hm, cfb, cm = carry
        g1 = (jnp.dot(hfb.astype(jnp.bfloat16), wfb,
                      preferred_element_type=jnp.float32) + xfb_s[k])
        g2 = (jnp.dot(hm.astype(jnp.bfloat16), wm,
                      preferred_element_type=jnp.float32) + xm_s[k])
        i1 = jnp.tanh(g1[:, 0:HID]) + 1.0
        f1 = jnp.tanh(g1[:, HID:2 * HID]) + 1.0
        gg1 = jnp.tanh(g1[:, 2 * HID:3 * HID])
        o1 = jnp.tanh(g1[:, 3 * HID:]) + 1.0
        i2 = jnp.tanh(g2[:, 0:HID]) + 1.0
        f2 = jnp.tanh(g2[:, HID:2 * HID]) + 1.0
        gg2 = jnp.tanh(g2[:, 2 * HID:3 * HID])
        o2 = jnp.tanh(g2[:, 3 * HID:]) + 1.0
        cfb2 = 0.5 * (f1 * cfb + i1 * gg1)
        cm2 = 0.5 * (f2 * cm + i2 * gg2)
        hfb2 = 0.5 * o1 * jnp.tanh(cfb2)
        hm2 = 0.5 * o2 * jnp.tanh(cm2)
        h_out[k, :, 0:HID] = hfb2
        h_out[k, :, HID:] = hm2
        return hfb2, hm2, cfb2, cm2

    def pair(p, carry):
        return step(2 * p + 1, step(2 * p, carry))

    npair = jnp.minimum(CH_R, LS - j * CH_R) // 2
    hfb, hm, cfb, cm = lax.fori_loop(
        0, npair, pair,
        (h_s[:, 0:HID], h_s[:, HID:], c_s[:, 0:HID], c_s[:, HID:]))
    h_s[:, 0:HID] = hfb
    h_s[:, HID:] = hm
    c_s[:, 0:HID] = cfb
    c_s[:, HID:] = cm


def _recurrence(ef, eb, em, w_in, b_fb, b_m, w_fb, w_m):
    return pl.pallas_call(
        _rec_body,
        grid=(GR_R,),
        in_specs=[
            pl.BlockSpec((CH_R, B, EMB), lambda j: (j, 0, 0)),
            pl.BlockSpec((CH_R, B, EMB), lambda j: (j, 0, 0)),
            pl.BlockSpec((CH_R, B, EMB), lambda j: (j, 0, 0)),
            pl.BlockSpec((EMB, 8 * HID), lambda j: (0, 0)),
            pl.BlockSpec((1, 1, 4 * HID), lambda j: (0, 0, 0)),
            pl.BlockSpec((1, 1, 4 * HID), lambda j: (0, 0, 0)),
            pl.BlockSpec((HID, 4 * HID), lambda j: (0, 0)),
            pl.BlockSpec((HID, 4 * HID), lambda j: (0, 0)),
        ],
        out_specs=pl.BlockSpec((CH_R, B, 2 * HID), lambda j: (j, 0, 0)),
        out_shape=jax.ShapeDtypeStruct((LS, B, 2 * HID), jnp.float32),
        scratch_shapes=[
            pltpu.VMEM((B, 2 * HID), jnp.float32),
            pltpu.VMEM((B, 2 * HID), jnp.float32),
            pltpu.VMEM((CH_R, B, 4 * HID), jnp.float32),
            pltpu.VMEM((CH_R, B, 4 * HID), jnp.float32),
        ],
    )(ef, eb, em, w_in, b_fb, b_m, w_fb, w_m)


def _proj_body(h_ref, hb_ref, ws_ref, bs_ref, wi_ref, bi_ref,
               len_ref, out_ref):
    j = pl.program_id(0)
    h = h_ref[...]
    hfb = jnp.concatenate([h[:, :, 0:HH], hb_ref[...]], axis=2)
    ctx = (hfb + h[:, :, 2 * HH:]).reshape(CH_P * B, HID)
    tloc = lax.broadcasted_iota(jnp.int32, (CH_P, B, 1), 0) + j * CH_P
    mask = tloc < (len_ref[...] - 1)
    for o, (w_ref, b_ref) in enumerate(((ws_ref, bs_ref), (wi_ref, bi_ref))):
        logits = jnp.dot(ctx, w_ref[...],
                         preferred_element_type=jnp.float32) + b_ref[...]
        m = jnp.max(logits, axis=1, keepdims=True)
        ls = jnp.log(jnp.sum(jnp.exp(logits - m), axis=1, keepdims=True)) + m
        out3 = (logits - ls).reshape(CH_P, B, NV)
        out_ref[o] = jnp.where(mask, out3, 0.0)


def _projection(h_all, hb, Wsub, bsub, Wins, bins, lengths):
    return pl.pallas_call(
        _proj_body,
        grid=(GR_P,),
        in_specs=[
            pl.BlockSpec((CH_P, B, 2 * HID), lambda j: (j, 0, 0)),
            pl.BlockSpec((CH_P, B, HH), lambda j: (j, 0, 0)),
            pl.BlockSpec((HID, NV), lambda j: (0, 0)),
            pl.BlockSpec((1, NV), lambda j: (0, 0)),
            pl.BlockSpec((HID, NV), lambda j: (0, 0)),
            pl.BlockSpec((1, NV), lambda j: (0, 0)),
            pl.BlockSpec((1, B, 1), lambda j: (0, 0, 0)),
        ],
        out_specs=pl.BlockSpec((2, CH_P, B, NV), lambda j: (0, j, 0, 0)),
        out_shape=jax.ShapeDtypeStruct((2, LT, B, NV), jnp.float32),
    )(h_all, hb, Wsub, bsub.reshape(1, -1), Wins, bins.reshape(1, -1),
      lengths.reshape(1, -1, 1))


def kernel(source_tokens, target_tokens, lengths, embedding,
           Wih_f, Whh_f, b_f, Wih_b, Whh_b, b_b,
           Wih_m, Whh_m, b_m, Wsub, bsub, Wins, bins):
    emb_pad = jnp.pad(embedding, ((0, TROWS - embedding.shape[0]), (0, 0)))
    table = _masked_table(emb_pad)

    idx_f = jnp.pad(source_tokens.reshape(-1), (0, NTOK - LS * B))
    idx_b = jnp.pad(jnp.flip(source_tokens, 0).reshape(-1),
                    (0, NTOK - LS * B))
    idx_m = jnp.pad(target_tokens.reshape(-1), (0, NTOK - LT * B))
    ef, eb, em = _sc_gather(table, idx_f, idx_b, idx_m)
    ef = ef.reshape(NTOK // B, B, EMB)
    eb = eb.reshape(NTOK // B, B, EMB)
    em = em.reshape(NTOK // B, B, EMB)

    def scale_ifo(a, gw):
        s = jnp.concatenate([jnp.full((2 * gw,), 0.5, jnp.float32),
                             jnp.ones((gw,), jnp.float32),
                             jnp.full((gw,), 0.5, jnp.float32)])
        return a * s

    w_in = jnp.concatenate([scale_ifo(Wih_f.T, HH), scale_ifo(Wih_b.T, HH),
                            scale_ifo(Wih_m.T, HID)], axis=1)
    b_fb = jnp.zeros((4 * HID,), jnp.float32)
    w_fb = jnp.zeros((HID, 4 * HID), jnp.float32)
    for sidx, (bias, whhT) in enumerate(((b_f, Whh_f.T), (b_b, Whh_b.T))):
        for gate in range(4):
            c0 = HID * gate + HH * sidx
            b_fb = b_fb.at[c0:c0 + HH].set(bias[gate * HH:(gate + 1) * HH])
            w_fb = w_fb.at[HH * sidx:HH * (sidx + 1), c0:c0 + HH].set(
                whhT[:, gate * HH:(gate + 1) * HH])
    b_fb = scale_ifo(b_fb, HID).reshape(1, 1, 4 * HID)
    b_m = scale_ifo(b_m, HID).reshape(1, 1, 4 * HID)
    w_fb = scale_ifo(w_fb, HID)
    w_m = scale_ifo(Whh_m.T, HID)

    h_all = _recurrence(ef, eb, em, w_in, b_fb, b_m,
                        w_fb.astype(jnp.bfloat16), w_m.astype(jnp.bfloat16))
    hb = jnp.flip(h_all[:, :, HH:2 * HH], 0)

    return _projection(h_all, hb, Wsub, bsub, Wins, bins, lengths)


# speedup vs baseline: 1.0097x; 1.0097x over previous
import functools

import jax
import jax.numpy as jnp
from jax import lax
from jax.experimental import pallas as pl
from jax.experimental.pallas import tpu as pltpu
from jax.experimental.pallas import tpu_sc as plsc

VOC = 512
EMB = 128
HID = 128
HH = HID // 2
B = 16
LS = 514
LT = 513
NV = VOC + 1

TROWS = 520
NW = 32
PER_W = 264
SUB = 88
NSUB = PER_W // SUB
NTOK = NW * PER_W

CH_R = 64
GR_R = 9
CH_P = 32
GR_P = 17

def _prep_body(emb_ref, t_ref):
    row = lax.broadcasted_iota(jnp.int32, (TROWS, 1), 0)
    t_ref[...] = jnp.where(row != 0, emb_ref[...], 0.0)


def _masked_table(emb_pad):
    return pl.pallas_call(
        _prep_body,
        out_shape=jax.ShapeDtypeStruct((TROWS, EMB), jnp.float32),
    )(emb_pad)


def _sc_gather(table, idx_f, idx_b, idx_m):
    mesh = plsc.VectorSubcoreMesh(core_axis_name="c", subcore_axis_name="s")

    @functools.partial(
        pl.kernel,
        mesh=mesh,
        out_type=[
            jax.ShapeDtypeStruct((NTOK, EMB), jnp.float32),
            jax.ShapeDtypeStruct((NTOK, EMB), jnp.float32),
            jax.ShapeDtypeStruct((NTOK, EMB), jnp.float32),
        ],
        scratch_types=[
            [pltpu.VMEM((SUB,), jnp.int32) for _ in range(3)],
            [pltpu.VMEM((SUB, EMB), jnp.float32) for _ in range(3)],
            pltpu.SemaphoreType.DMA,
            pltpu.SemaphoreType.DMA,
        ],
    )
    def k(t_hbm, if_hbm, ib_hbm, im_hbm,
          xf_hbm, xb_hbm, xm_hbm, idx_v, row_v, gsem, osem):
        wid = lax.axis_index("s") * 2 + lax.axis_index("c")
        base = wid * PER_W
        tasks = []
        for s in range(NSUB):
            off = base + s * SUB
            tasks.append((if_hbm, xf_hbm, off))
            tasks.append((ib_hbm, xb_hbm, off))
            tasks.append((im_hbm, xm_hbm, off))

        def start(t, slot):
            src_idx, _, off = tasks[t]
            pltpu.sync_copy(src_idx.at[pl.ds(off, SUB)], idx_v[slot])
            return pltpu.async_copy(t_hbm.at[idx_v[slot]], row_v[slot], gsem)

        outs = []
        g = start(0, 0)
        for t in range(len(tasks)):
            slot = t % 3
            g.wait()
            if t + 1 < len(tasks):
                g = start(t + 1, (t + 1) % 3)
            _, dst, off = tasks[t]
            o = pltpu.async_copy(row_v[slot], dst.at[pl.ds(off, SUB)], osem)
            outs.append(o)
            if len(outs) >= 2:
                outs.pop(0).wait()
        for o in outs:
            o.wait()

    return k(table, idx_f, idx_b, idx_m)


def _rec_body(ef_ref, eb_ref, em_ref, wi_ref, bfb_ref, bm_ref,
              wfb_ref, wm_ref, h_out, h_s, c_s, xfb_s, xm_s):
    j = pl.program_id(0)

    @pl.when(j == 0)
    def _():
        h_s[...] = jnp.zeros_like(h_s)
        c_s[...] = jnp.zeros_like(c_s)

    for sidx, e_ref in ((0, ef_ref), (1, eb_ref)):
        emb = e_ref[...].reshape(CH_R * B, EMB)
        xs = jnp.dot(emb, wi_ref[:, 4 * HH * sidx:4 * HH * (sidx + 1)],
                     preferred_element_type=jnp.float32)
        for gate in range(4):
            c0 = HID * gate + HH * sidx
            xfb_s[:, :, c0:c0 + HH] = (
                xs[:, gate * HH:(gate + 1) * HH].reshape(CH_R, B, HH)
                + bfb_ref[:, :, c0:c0 + HH])
    emb = em_ref[...].reshape(CH_R * B, EMB)
    xm_s[...] = (jnp.dot(emb, wi_ref[:, 8 * HH:],
                         preferred_element_type=jnp.float32)
                 + bm_ref[0]).reshape(CH_R, B, 4 * HID)

    wfb = wfb_ref[...]
    wm = wm_ref[...]

    def step(k, carry):
        hfb, hm, cfb, cm = carry
        g1 = (jnp.dot(hfb.astype(jnp.bfloat16), wfb,
                      preferred_element_type=jnp.float32) + xfb_s[k])
        g2 = (jnp.dot(hm.astype(jnp.bfloat16), wm,
                      preferred_element_type=jnp.float32) + xm_s[k])
        i1 = jax.nn.sigmoid(g1[:, 0:HID])
        f1 = jax.nn.sigmoid(g1[:, HID:2 * HID])
        gg1 = jnp.tanh(g1[:, 2 * HID:3 * HID])
        o1 = jax.nn.sigmoid(g1[:, 3 * HID:])
        i2 = jax.nn.sigmoid(g2[:, 0:HID])
        f2 = jax.nn.sigmoid(g2[:, HID:2 * HID])
        gg2 = jnp.tanh(g2[:, 2 * HID:3 * HID])
        o2 = jax.nn.sigmoid(g2[:, 3 * HID:])
        cfb2 = f1 * cfb + i1 * gg1
        cm2 = f2 * cm + i2 * gg2
        hfb2 = o1 * jnp.tanh(cfb2)
        hm2 = o2 * jnp.tanh(cm2)
        h_out[k, :, 0:HID] = hfb2
        h_out[k, :, HID:] = hm2
        return hfb2, hm2, cfb2, cm2

    def pair(p, carry):
        return step(2 * p + 1, step(2 * p, carry))

    npair = jnp.minimum(CH_R, LS - j * CH_R) // 2
    hfb, hm, cfb, cm = lax.fori_loop(
        0, npair, pair,
        (h_s[:, 0:HID], h_s[:, HID:], c_s[:, 0:HID], c_s[:, HID:]))
    h_s[:, 0:HID] = hfb
    h_s[:, HID:] = hm
    c_s[:, 0:HID] = cfb
    c_s[:, HID:] = cm


def _recurrence(ef, eb, em, w_in, b_fb, b_m, w_fb, w_m):
    return pl.pallas_call(
        _rec_body,
        grid=(GR_R,),
        in_specs=[
            pl.BlockSpec((CH_R, B, EMB), lambda j: (j, 0, 0)),
            pl.BlockSpec((CH_R, B, EMB), lambda j: (j, 0, 0)),
            pl.BlockSpec((CH_R, B, EMB), lambda j: (j, 0, 0)),
            pl.BlockSpec((EMB, 8 * HID), lambda j: (0, 0)),
            pl.BlockSpec((1, 1, 4 * HID), lambda j: (0, 0, 0)),
            pl.BlockSpec((1, 1, 4 * HID), lambda j: (0, 0, 0)),
            pl.BlockSpec((HID, 4 * HID), lambda j: (0, 0)),
            pl.BlockSpec((HID, 4 * HID), lambda j: (0, 0)),
        ],
        out_specs=pl.BlockSpec((CH_R, B, 2 * HID), lambda j: (j, 0, 0)),
        out_shape=jax.ShapeDtypeStruct((LS, B, 2 * HID), jnp.float32),
        scratch_shapes=[
            pltpu.VMEM((B, 2 * HID), jnp.float32),
            pltpu.VMEM((B, 2 * HID), jnp.float32),
            pltpu.VMEM((CH_R, B, 4 * HID), jnp.float32),
            pltpu.VMEM((CH_R, B, 4 * HID), jnp.float32),
        ],
    )(ef, eb, em, w_in, b_fb, b_m, w_fb, w_m)


def _proj_body(h_ref, hb_ref, ws_ref, bs_ref, wi_ref, bi_ref,
               len_ref, out_ref):
    j = pl.program_id(0)
    h = h_ref[...]
    hfb = jnp.concatenate([h[:, :, 0:HH], hb_ref[...]], axis=2)
    ctx = (hfb + h[:, :, 2 * HH:]).reshape(CH_P * B, HID)
    tloc = lax.broadcasted_iota(jnp.int32, (CH_P, B, 1), 0) + j * CH_P
    mask = tloc < (len_ref[...] - 1)
    for o, (w_ref, b_ref) in enumerate(((ws_ref, bs_ref), (wi_ref, bi_ref))):
        logits = jnp.dot(ctx, w_ref[...],
                         preferred_element_type=jnp.float32) + b_ref[...]
        m = jnp.max(logits, axis=1, keepdims=True)
        ls = jnp.log(jnp.sum(jnp.exp(logits - m), axis=1, keepdims=True)) + m
        out3 = (logits - ls).reshape(CH_P, B, NV)
        out_ref[o] = jnp.where(mask, out3, 0.0)


def _projection(h_all, hb, Wsub, bsub, Wins, bins, lengths):
    return pl.pallas_call(
        _proj_body,
        grid=(GR_P,),
        in_specs=[
            pl.BlockSpec((CH_P, B, 2 * HID), lambda j: (j, 0, 0)),
            pl.BlockSpec((CH_P, B, HH), lambda j: (j, 0, 0)),
            pl.BlockSpec((HID, NV), lambda j: (0, 0)),
            pl.BlockSpec((1, NV), lambda j: (0, 0)),
            pl.BlockSpec((HID, NV), lambda j: (0, 0)),
            pl.BlockSpec((1, NV), lambda j: (0, 0)),
            pl.BlockSpec((1, B, 1), lambda j: (0, 0, 0)),
        ],
        out_specs=pl.BlockSpec((2, CH_P, B, NV), lambda j: (0, j, 0, 0)),
        out_shape=jax.ShapeDtypeStruct((2, LT, B, NV), jnp.float32),
    )(h_all, hb, Wsub, bsub.reshape(1, -1), Wins, bins.reshape(1, -1),
      lengths.reshape(1, -1, 1))


def kernel(source_tokens, target_tokens, lengths, embedding,
           Wih_f, Whh_f, b_f, Wih_b, Whh_b, b_b,
           Wih_m, Whh_m, b_m, Wsub, bsub, Wins, bins):
    emb_pad = jnp.pad(embedding, ((0, TROWS - embedding.shape[0]), (0, 0)))
    table = _masked_table(emb_pad)

    idx_f = jnp.pad(source_tokens.reshape(-1), (0, NTOK - LS * B))
    idx_b = jnp.pad(jnp.flip(source_tokens, 0).reshape(-1),
                    (0, NTOK - LS * B))
    idx_m = jnp.pad(target_tokens.reshape(-1), (0, NTOK - LT * B))
    ef, eb, em = _sc_gather(table, idx_f, idx_b, idx_m)
    ef = ef.reshape(NTOK // B, B, EMB)
    eb = eb.reshape(NTOK // B, B, EMB)
    em = em.reshape(NTOK // B, B, EMB)

    w_in = jnp.concatenate([Wih_f.T, Wih_b.T, Wih_m.T], axis=1)
    b_fb = jnp.zeros((4 * HID,), jnp.float32)
    w_fb = jnp.zeros((HID, 4 * HID), jnp.float32)
    for sidx, (bias, whhT) in enumerate(((b_f, Whh_f.T), (b_b, Whh_b.T))):
        for gate in range(4):
            c0 = HID * gate + HH * sidx
            b_fb = b_fb.at[c0:c0 + HH].set(bias[gate * HH:(gate + 1) * HH])
            w_fb = w_fb.at[HH * sidx:HH * (sidx + 1), c0:c0 + HH].set(
                whhT[:, gate * HH:(gate + 1) * HH])
    b_fb = b_fb.reshape(1, 1, 4 * HID)
    b_m = b_m.reshape(1, 1, 4 * HID)

    h_all = _recurrence(ef, eb, em, w_in, b_fb, b_m,
                        w_fb.astype(jnp.bfloat16),
                        Whh_m.T.astype(jnp.bfloat16))
    hb = jnp.flip(h_all[:, :, HH:2 * HH], 0)

    return _projection(h_all, hb, Wsub, bsub, Wins, bins, lengths)


# speedup vs baseline: 1.0403x; 1.0303x over previous
import functools

import jax
import jax.numpy as jnp
from jax import lax
from jax.experimental import pallas as pl
from jax.experimental.pallas import tpu as pltpu
from jax.experimental.pallas import tpu_sc as plsc

VOC = 512
EMB = 128
HID = 128
HH = HID // 2
B = 16
LS = 514
LT = 513
NV = VOC + 1

TROWS = 520
NW = 32
PER_W = 264
SUB = 88
NSUB = PER_W // SUB
NTOK = NW * PER_W

CH_R = 64
GR_R = 9
CH_P = 32
GR_P = 17

def _prep_body(emb_ref, t_ref):
    row = lax.broadcasted_iota(jnp.int32, (TROWS, 1), 0)
    t_ref[...] = jnp.where(row != 0, emb_ref[...], 0.0)


def _masked_table(emb_pad):
    return pl.pallas_call(
        _prep_body,
        out_shape=jax.ShapeDtypeStruct((TROWS, EMB), jnp.float32),
    )(emb_pad)


def _sc_gather(table, idx_f, idx_b, idx_m):
    mesh = plsc.VectorSubcoreMesh(core_axis_name="c", subcore_axis_name="s")

    @functools.partial(
        pl.kernel,
        mesh=mesh,
        out_type=[
            jax.ShapeDtypeStruct((NTOK, EMB), jnp.float32),
            jax.ShapeDtypeStruct((NTOK, EMB), jnp.float32),
            jax.ShapeDtypeStruct((NTOK, EMB), jnp.float32),
        ],
        scratch_types=[
            [pltpu.VMEM((SUB,), jnp.int32) for _ in range(4)],
            [pltpu.VMEM((SUB, EMB), jnp.float32) for _ in range(4)],
            pltpu.SemaphoreType.DMA,
            pltpu.SemaphoreType.DMA,
        ],
    )
    def k(t_hbm, if_hbm, ib_hbm, im_hbm,
          xf_hbm, xb_hbm, xm_hbm, idx_v, row_v, gsem, osem):
        wid = lax.axis_index("s") * 2 + lax.axis_index("c")
        base = wid * PER_W
        tasks = []
        for s in range(NSUB):
            off = base + s * SUB
            tasks.append((if_hbm, xf_hbm, off))
            tasks.append((ib_hbm, xb_hbm, off))
            tasks.append((im_hbm, xm_hbm, off))
        nt = len(tasks)

        def start(t):
            src_idx, _, off = tasks[t]
            slot = t % 4
            pltpu.sync_copy(src_idx.at[pl.ds(off, SUB)], idx_v[slot])
            return pltpu.async_copy(t_hbm.at[idx_v[slot]], row_v[slot], gsem)

        gath = [start(0), start(1)]
        outs = []
        for t in range(nt):
            gath[t].wait()
            if len(outs) >= 2:
                outs.pop(0).wait()
            if t + 2 < nt:
                gath.append(start(t + 2))
            _, dst, off = tasks[t]
            outs.append(
                pltpu.async_copy(row_v[t % 4], dst.at[pl.ds(off, SUB)], osem))
        for o in outs:
            o.wait()

    return k(table, idx_f, idx_b, idx_m)


def _rec_body(ef_ref, eb_ref, em_ref, wi_ref, bfb_ref, bm_ref,
              wfb_ref, wm_ref, h_out, h_s, c_s, xfb_s, xm_s):
    j = pl.program_id(0)

    @pl.when(j == 0)
    def _():
        h_s[...] = jnp.zeros_like(h_s)
        c_s[...] = jnp.zeros_like(c_s)

    for sidx, e_ref in ((0, ef_ref), (1, eb_ref)):
        emb = e_ref[...].reshape(CH_R * B, EMB)
        xs = jnp.dot(emb, wi_ref[:, 4 * HH * sidx:4 * HH * (sidx + 1)],
                     preferred_element_type=jnp.float32)
        for gate in range(4):
            c0 = HID * gate + HH * sidx
            xfb_s[:, :, c0:c0 + HH] = (
                xs[:, gate * HH:(gate + 1) * HH].reshape(CH_R, B, HH)
                + bfb_ref[:, :, c0:c0 + HH])
    emb = em_ref[...].reshape(CH_R * B, EMB)
    xm_s[...] = (jnp.dot(emb, wi_ref[:, 8 * HH:],
                         preferred_element_type=jnp.float32)
                 + bm_ref[0]).reshape(CH_R, B, 4 * HID)

    wfb = wfb_ref[...]
    wm = wm_ref[...]

    def step(k, carry):
        hfb, hm, cfb, cm = carry
        g1 = (jnp.dot(hfb.astype(jnp.bfloat16), wfb,
                      preferred_element_type=jnp.float32) + xfb_s[k])
        g2 = (jnp.dot(hm.astype(jnp.bfloat16), wm,
                      preferred_element_type=jnp.float32) + xm_s[k])
        i1 = jax.nn.sigmoid(g1[:, 0:HID])
        f1 = jax.nn.sigmoid(g1[:, HID:2 * HID])
        gg1 = jnp.tanh(g1[:, 2 * HID:3 * HID])
        o1 = jax.nn.sigmoid(g1[:, 3 * HID:])
        i2 = jax.nn.sigmoid(g2[:, 0:HID])
        f2 = jax.nn.sigmoid(g2[:, HID:2 * HID])
        gg2 = jnp.tanh(g2[:, 2 * HID:3 * HID])
        o2 = jax.nn.sigmoid(g2[:, 3 * HID:])
        cfb2 = f1 * cfb + i1 * gg1
        cm2 = f2 * cm + i2 * gg2
        hfb2 = o1 * jnp.tanh(cfb2)
        hm2 = o2 * jnp.tanh(cm2)
        h_out[k, :, 0:HID] = hfb2
        h_out[k, :, HID:] = hm2
        return hfb2, hm2, cfb2, cm2

    def pair(p, carry):
        return step(2 * p + 1, step(2 * p, carry))

    npair = jnp.minimum(CH_R, LS - j * CH_R) // 2
    hfb, hm, cfb, cm = lax.fori_loop(
        0, npair, pair,
        (h_s[:, 0:HID], h_s[:, HID:], c_s[:, 0:HID], c_s[:, HID:]))
    h_s[:, 0:HID] = hfb
    h_s[:, HID:] = hm
    c_s[:, 0:HID] = cfb
    c_s[:, HID:] = cm


def _recurrence(ef, eb, em, w_in, b_fb, b_m, w_fb, w_m):
    return pl.pallas_call(
        _rec_body,
        grid=(GR_R,),
        in_specs=[
            pl.BlockSpec((CH_R, B, EMB), lambda j: (j, 0, 0)),
            pl.BlockSpec((CH_R, B, EMB), lambda j: (j, 0, 0)),
            pl.BlockSpec((CH_R, B, EMB), lambda j: (j, 0, 0)),
            pl.BlockSpec((EMB, 8 * HID), lambda j: (0, 0)),
            pl.BlockSpec((1, 1, 4 * HID), lambda j: (0, 0, 0)),
            pl.BlockSpec((1, 1, 4 * HID), lambda j: (0, 0, 0)),
            pl.BlockSpec((HID, 4 * HID), lambda j: (0, 0)),
            pl.BlockSpec((HID, 4 * HID), lambda j: (0, 0)),
        ],
        out_specs=pl.BlockSpec((CH_R, B, 2 * HID), lambda j: (j, 0, 0)),
        out_shape=jax.ShapeDtypeStruct((LS, B, 2 * HID), jnp.float32),
        scratch_shapes=[
            pltpu.VMEM((B, 2 * HID), jnp.float32),
            pltpu.VMEM((B, 2 * HID), jnp.float32),
            pltpu.VMEM((CH_R, B, 4 * HID), jnp.float32),
            pltpu.VMEM((CH_R, B, 4 * HID), jnp.float32),
        ],
    )(ef, eb, em, w_in, b_fb, b_m, w_fb, w_m)


def _proj_body(h_ref, hb_ref, ws_ref, bs_ref, wi_ref, bi_ref,
               len_ref, out_ref):
    j = pl.program_id(0)
    h = h_ref[...]
    hfb = jnp.concatenate([h[:, :, 0:HH], hb_ref[...]], axis=2)
    ctx = (hfb + h[:, :, 2 * HH:]).reshape(CH_P * B, HID)
    tloc = lax.broadcasted_iota(jnp.int32, (CH_P, B, 1), 0) + j * CH_P
    mask = tloc < (len_ref[...] - 1)
    for o, (w_ref, b_ref) in enumerate(((ws_ref, bs_ref), (wi_ref, bi_ref))):
        logits = jnp.dot(ctx, w_ref[...],
                         preferred_element_type=jnp.float32) + b_ref[...]
        m = jnp.max(logits, axis=1, keepdims=True)
        ls = jnp.log(jnp.sum(jnp.exp(logits - m), axis=1, keepdims=True)) + m
        out3 = (logits - ls).reshape(CH_P, B, NV)
        out_ref[o] = jnp.where(mask, out3, 0.0)


def _projection(h_all, hb, Wsub, bsub, Wins, bins, lengths):
    return pl.pallas_call(
        _proj_body,
        grid=(GR_P,),
        in_specs=[
            pl.BlockSpec((CH_P, B, 2 * HID), lambda j: (j, 0, 0)),
            pl.BlockSpec((CH_P, B, HH), lambda j: (j, 0, 0)),
            pl.BlockSpec((HID, NV), lambda j: (0, 0)),
            pl.BlockSpec((1, NV), lambda j: (0, 0)),
            pl.BlockSpec((HID, NV), lambda j: (0, 0)),
            pl.BlockSpec((1, NV), lambda j: (0, 0)),
            pl.BlockSpec((1, B, 1), lambda j: (0, 0, 0)),
        ],
        out_specs=pl.BlockSpec((2, CH_P, B, NV), lambda j: (0, j, 0, 0)),
        out_shape=jax.ShapeDtypeStruct((2, LT, B, NV), jnp.float32),
    )(h_all, hb, Wsub, bsub.reshape(1, -1), Wins, bins.reshape(1, -1),
      lengths.reshape(1, -1, 1))


def kernel(source_tokens, target_tokens, lengths, embedding,
           Wih_f, Whh_f, b_f, Wih_b, Whh_b, b_b,
           Wih_m, Whh_m, b_m, Wsub, bsub, Wins, bins):
    emb_pad = jnp.pad(embedding, ((0, TROWS - embedding.shape[0]), (0, 0)))
    table = _masked_table(emb_pad)

    idx_f = jnp.pad(source_tokens.reshape(-1), (0, NTOK - LS * B))
    idx_b = jnp.pad(jnp.flip(source_tokens, 0).reshape(-1),
                    (0, NTOK - LS * B))
    idx_m = jnp.pad(target_tokens.reshape(-1), (0, NTOK - LT * B))
    ef, eb, em = _sc_gather(table, idx_f, idx_b, idx_m)
    ef = ef.reshape(NTOK // B, B, EMB)
    eb = eb.reshape(NTOK // B, B, EMB)
    em = em.reshape(NTOK // B, B, EMB)

    w_in = jnp.concatenate([Wih_f.T, Wih_b.T, Wih_m.T], axis=1)
    b_fb = jnp.zeros((4 * HID,), jnp.float32)
    w_fb = jnp.zeros((HID, 4 * HID), jnp.float32)
    for sidx, (bias, whhT) in enumerate(((b_f, Whh_f.T), (b_b, Whh_b.T))):
        for gate in range(4):
            c0 = HID * gate + HH * sidx
            b_fb = b_fb.at[c0:c0 + HH].set(bias[gate * HH:(gate + 1) * HH])
            w_fb = w_fb.at[HH * sidx:HH * (sidx + 1), c0:c0 + HH].set(
                whhT[:, gate * HH:(gate + 1) * HH])
    b_fb = b_fb.reshape(1, 1, 4 * HID)
    b_m = b_m.reshape(1, 1, 4 * HID)

    h_all = _recurrence(ef, eb, em, w_in, b_fb, b_m,
                        w_fb.astype(jnp.bfloat16),
                        Whh_m.T.astype(jnp.bfloat16))
    hb = jnp.flip(h_all[:, :, HH:2 * HH], 0)

    return _projection(h_all, hb, Wsub, bsub, Wins, bins, lengths)


# speedup vs baseline: 1.0449x; 1.0045x over previous
import functools

import jax
import jax.numpy as jnp
from jax import lax
from jax.experimental import pallas as pl
from jax.experimental.pallas import tpu as pltpu
from jax.experimental.pallas import tpu_sc as plsc

VOC = 512
EMB = 128
HID = 128
HH = HID // 2
B = 16
LS = 514
LT = 513
NV = VOC + 1

TROWS = 520
NW = 32
PER_W = 264
SUB = 88
NSUB = PER_W // SUB
NTOK = NW * PER_W

CH_R = 128
GR_R = 5
CH_P = 32
GR_P = 17

def _prep_body(emb_ref, t_ref):
    row = lax.broadcasted_iota(jnp.int32, (TROWS, 1), 0)
    t_ref[...] = jnp.where(row != 0, emb_ref[...], 0.0)


def _masked_table(emb_pad):
    return pl.pallas_call(
        _prep_body,
        out_shape=jax.ShapeDtypeStruct((TROWS, EMB), jnp.float32),
    )(emb_pad)


def _sc_gather(table, idx_f, idx_b, idx_m):
    mesh = plsc.VectorSubcoreMesh(core_axis_name="c", subcore_axis_name="s")

    @functools.partial(
        pl.kernel,
        mesh=mesh,
        out_type=[
            jax.ShapeDtypeStruct((NTOK, EMB), jnp.float32),
            jax.ShapeDtypeStruct((NTOK, EMB), jnp.float32),
            jax.ShapeDtypeStruct((NTOK, EMB), jnp.float32),
        ],
        scratch_types=[
            [pltpu.VMEM((SUB,), jnp.int32) for _ in range(4)],
            [pltpu.VMEM((SUB, EMB), jnp.float32) for _ in range(4)],
            pltpu.SemaphoreType.DMA,
            pltpu.SemaphoreType.DMA,
        ],
    )
    def k(t_hbm, if_hbm, ib_hbm, im_hbm,
          xf_hbm, xb_hbm, xm_hbm, idx_v, row_v, gsem, osem):
        wid = lax.axis_index("s") * 2 + lax.axis_index("c")
        base = wid * PER_W
        tasks = []
        for s in range(NSUB):
            off = base + s * SUB
            tasks.append((if_hbm, xf_hbm, off))
            tasks.append((ib_hbm, xb_hbm, off))
            tasks.append((im_hbm, xm_hbm, off))
        nt = len(tasks)

        def start(t):
            src_idx, _, off = tasks[t]
            slot = t % 4
            pltpu.sync_copy(src_idx.at[pl.ds(off, SUB)], idx_v[slot])
            return pltpu.async_copy(t_hbm.at[idx_v[slot]], row_v[slot], gsem)

        gath = [start(0), start(1)]
        outs = []
        for t in range(nt):
            gath[t].wait()
            if len(outs) >= 2:
                outs.pop(0).wait()
            if t + 2 < nt:
                gath.append(start(t + 2))
            _, dst, off = tasks[t]
            outs.append(
                pltpu.async_copy(row_v[t % 4], dst.at[pl.ds(off, SUB)], osem))
        for o in outs:
            o.wait()

    return k(table, idx_f, idx_b, idx_m)


def _rec_body(ef_ref, eb_ref, em_ref, wi_ref, bfb_ref, bm_ref,
              wfb_ref, wm_ref, h_out, h_s, c_s, xfb_s, xm_s):
    j = pl.program_id(0)

    @pl.when(j == 0)
    def _():
        h_s[...] = jnp.zeros_like(h_s)
        c_s[...] = jnp.zeros_like(c_s)

    for sidx, e_ref in ((0, ef_ref), (1, eb_ref)):
        emb = e_ref[...].reshape(CH_R * B, EMB)
        xs = jnp.dot(emb, wi_ref[:, 4 * HH * sidx:4 * HH * (sidx + 1)],
                     preferred_element_type=jnp.float32)
        for gate in range(4):
            c0 = HID * gate + HH * sidx
            xfb_s[:, :, c0:c0 + HH] = (
                xs[:, gate * HH:(gate + 1) * HH].reshape(CH_R, B, HH)
                + bfb_ref[:, :, c0:c0 + HH])
    emb = em_ref[...].reshape(CH_R * B, EMB)
    xm_s[...] = (jnp.dot(emb, wi_ref[:, 8 * HH:],
                         preferred_element_type=jnp.float32)
                 + bm_ref[0]).reshape(CH_R, B, 4 * HID)

    wfb = wfb_ref[...]
    wm = wm_ref[...]

    def step(k, carry):
        hfb, hm, cfb, cm = carry
        g1 = (jnp.dot(hfb.astype(jnp.bfloat16), wfb,
                      preferred_element_type=jnp.float32) + xfb_s[k])
        g2 = (jnp.dot(hm.astype(jnp.bfloat16), wm,
                      preferred_element_type=jnp.float32) + xm_s[k])
        i1 = jax.nn.sigmoid(g1[:, 0:HID])
        f1 = jax.nn.sigmoid(g1[:, HID:2 * HID])
        gg1 = jnp.tanh(g1[:, 2 * HID:3 * HID])
        o1 = jax.nn.sigmoid(g1[:, 3 * HID:])
        i2 = jax.nn.sigmoid(g2[:, 0:HID])
        f2 = jax.nn.sigmoid(g2[:, HID:2 * HID])
        gg2 = jnp.tanh(g2[:, 2 * HID:3 * HID])
        o2 = jax.nn.sigmoid(g2[:, 3 * HID:])
        cfb2 = f1 * cfb + i1 * gg1
        cm2 = f2 * cm + i2 * gg2
        hfb2 = o1 * jnp.tanh(cfb2)
        hm2 = o2 * jnp.tanh(cm2)
        h_out[k, :, 0:HID] = hfb2
        h_out[k, :, HID:] = hm2
        return hfb2, hm2, cfb2, cm2

    def pair(p, carry):
        return step(2 * p + 1, step(2 * p, carry))

    npair = jnp.minimum(CH_R, LS - j * CH_R) // 2
    hfb, hm, cfb, cm = lax.fori_loop(
        0, npair, pair,
        (h_s[:, 0:HID], h_s[:, HID:], c_s[:, 0:HID], c_s[:, HID:]))
    h_s[:, 0:HID] = hfb
    h_s[:, HID:] = hm
    c_s[:, 0:HID] = cfb
    c_s[:, HID:] = cm


def _recurrence(ef, eb, em, w_in, b_fb, b_m, w_fb, w_m):
    return pl.pallas_call(
        _rec_body,
        grid=(GR_R,),
        in_specs=[
            pl.BlockSpec((CH_R, B, EMB), lambda j: (j, 0, 0)),
            pl.BlockSpec((CH_R, B, EMB), lambda j: (j, 0, 0)),
            pl.BlockSpec((CH_R, B, EMB), lambda j: (j, 0, 0)),
            pl.BlockSpec((EMB, 8 * HID), lambda j: (0, 0)),
            pl.BlockSpec((1, 1, 4 * HID), lambda j: (0, 0, 0)),
            pl.BlockSpec((1, 1, 4 * HID), lambda j: (0, 0, 0)),
            pl.BlockSpec((HID, 4 * HID), lambda j: (0, 0)),
            pl.BlockSpec((HID, 4 * HID), lambda j: (0, 0)),
        ],
        out_specs=pl.BlockSpec((CH_R, B, 2 * HID), lambda j: (j, 0, 0)),
        out_shape=jax.ShapeDtypeStruct((LS, B, 2 * HID), jnp.float32),
        scratch_shapes=[
            pltpu.VMEM((B, 2 * HID), jnp.float32),
            pltpu.VMEM((B, 2 * HID), jnp.float32),
            pltpu.VMEM((CH_R, B, 4 * HID), jnp.float32),
            pltpu.VMEM((CH_R, B, 4 * HID), jnp.float32),
        ],
    )(ef, eb, em, w_in, b_fb, b_m, w_fb, w_m)


def _proj_body(h_ref, hb_ref, ws_ref, bs_ref, wi_ref, bi_ref,
               len_ref, out_ref):
    j = pl.program_id(0)
    h = h_ref[...]
    hfb = jnp.concatenate([h[:, :, 0:HH], hb_ref[...]], axis=2)
    ctx = (hfb + h[:, :, 2 * HH:]).reshape(CH_P * B, HID)
    tloc = lax.broadcasted_iota(jnp.int32, (CH_P, B, 1), 0) + j * CH_P
    mask = tloc < (len_ref[...] - 1)
    for o, (w_ref, b_ref) in enumerate(((ws_ref, bs_ref), (wi_ref, bi_ref))):
        logits = jnp.dot(ctx, w_ref[...],
                         preferred_element_type=jnp.float32) + b_ref[...]
        m = jnp.max(logits, axis=1, keepdims=True)
        ls = jnp.log(jnp.sum(jnp.exp(logits - m), axis=1, keepdims=True)) + m
        out3 = (logits - ls).reshape(CH_P, B, NV)
        out_ref[o] = jnp.where(mask, out3, 0.0)


def _projection(h_all, hb, Wsub, bsub, Wins, bins, lengths):
    return pl.pallas_call(
        _proj_body,
        grid=(GR_P,),
        in_specs=[
            pl.BlockSpec((CH_P, B, 2 * HID), lambda j: (j, 0, 0)),
            pl.BlockSpec((CH_P, B, HH), lambda j: (j, 0, 0)),
            pl.BlockSpec((HID, NV), lambda j: (0, 0)),
            pl.BlockSpec((1, NV), lambda j: (0, 0)),
            pl.BlockSpec((HID, NV), lambda j: (0, 0)),
            pl.BlockSpec((1, NV), lambda j: (0, 0)),
            pl.BlockSpec((1, B, 1), lambda j: (0, 0, 0)),
        ],
        out_specs=pl.BlockSpec((2, CH_P, B, NV), lambda j: (0, j, 0, 0)),
        out_shape=jax.ShapeDtypeStruct((2, LT, B, NV), jnp.float32),
    )(h_all, hb, Wsub, bsub.reshape(1, -1), Wins, bins.reshape(1, -1),
      lengths.reshape(1, -1, 1))


def kernel(source_tokens, target_tokens, lengths, embedding,
           Wih_f, Whh_f, b_f, Wih_b, Whh_b, b_b,
           Wih_m, Whh_m, b_m, Wsub, bsub, Wins, bins):
    emb_pad = jnp.pad(embedding, ((0, TROWS - embedding.shape[0]), (0, 0)))
    table = _masked_table(emb_pad)

    idx_f = jnp.pad(source_tokens.reshape(-1), (0, NTOK - LS * B))
    idx_b = jnp.pad(jnp.flip(source_tokens, 0).reshape(-1),
                    (0, NTOK - LS * B))
    idx_m = jnp.pad(target_tokens.reshape(-1), (0, NTOK - LT * B))
    ef, eb, em = _sc_gather(table, idx_f, idx_b, idx_m)
    ef = ef.reshape(NTOK // B, B, EMB)
    eb = eb.reshape(NTOK // B, B, EMB)
    em = em.reshape(NTOK // B, B, EMB)

    w_in = jnp.concatenate([Wih_f.T, Wih_b.T, Wih_m.T], axis=1)
    b_fb = jnp.zeros((4 * HID,), jnp.float32)
    w_fb = jnp.zeros((HID, 4 * HID), jnp.float32)
    for sidx, (bias, whhT) in enumerate(((b_f, Whh_f.T), (b_b, Whh_b.T))):
        for gate in range(4):
            c0 = HID * gate + HH * sidx
            b_fb = b_fb.at[c0:c0 + HH].set(bias[gate * HH:(gate + 1) * HH])
            w_fb = w_fb.at[HH * sidx:HH * (sidx + 1), c0:c0 + HH].set(
                whhT[:, gate * HH:(gate + 1) * HH])
    b_fb = b_fb.reshape(1, 1, 4 * HID)
    b_m = b_m.reshape(1, 1, 4 * HID)

    h_all = _recurrence(ef, eb, em, w_in, b_fb, b_m,
                        w_fb.astype(jnp.bfloat16),
                        Whh_m.T.astype(jnp.bfloat16))
    hb = jnp.flip(h_all[:, :, HH:2 * HH], 0)

    return _projection(h_all, hb, Wsub, bsub, Wins, bins, lengths)
